# baseline (speedup 1.0000x reference)
import functools

import jax
import jax.numpy as jnp
from jax import lax
from jax.experimental import pallas as pl
from jax.experimental.pallas import tpu as pltpu
from jax.experimental.pallas import tpu_sc as plsc


def _sc_add(B, S, D):
    NC, NS = 2, 16
    NW = NC * NS
    SW = S // NW
    C = 16
    CHW = C * D
    n_chunks = SW // C

    mesh = plsc.VectorSubcoreMesh(core_axis_name="c", subcore_axis_name="s")

    @functools.partial(
        pl.kernel,
        mesh=mesh,
        out_type=jax.ShapeDtypeStruct((B * S * D,), jnp.float32),
        scratch_types=[
            pltpu.VMEM((CHW,), jnp.float32),
            pltpu.VMEM((CHW,), jnp.float32),
        ],
    )
    def run(x_hbm, pos_hbm, out_hbm, p_v, x_v):
        wid = lax.axis_index("s") * NC + lax.axis_index("c")
        s_base = wid * SW

        def chunk_body(c, carry):
            s0 = s_base + c * C
            pltpu.sync_copy(pos_hbm.at[pl.ds(s0 * D, CHW)], p_v)
            for b in range(B):
                row0 = (b * S) * D + s0 * D
                pltpu.sync_copy(x_hbm.at[pl.ds(row0, CHW)], x_v)

                def add16(i, acc):
                    off = i * 16
                    x_v[pl.ds(off, 16)] = x_v[pl.ds(off, 16)] + p_v[pl.ds(off, 16)]
                    return acc

                lax.fori_loop(0, CHW // 16, add16, 0)
                pltpu.sync_copy(x_v, out_hbm.at[pl.ds(row0, CHW)])
            return carry

        lax.fori_loop(0, n_chunks, chunk_body, 0)

    return run


def kernel(x, pos_weight):
    B, S, D = x.shape
    out = _sc_add(B, S, D)(x.reshape(-1), pos_weight[:S].reshape(-1))
    return out.reshape(B, S, D)


# speedup vs baseline: 1.5955x; 1.5955x over previous
import functools

import jax
import jax.numpy as jnp
from jax import lax
from jax.experimental import pallas as pl
from jax.experimental.pallas import tpu as pltpu
from jax.experimental.pallas import tpu_sc as plsc


def _sc_add(B, S, D):
    NC, NS = 2, 16
    NW = NC * NS
    SW = S // NW
    C = 16
    CHW = C * D
    n_chunks = SW // C
    n_steps = n_chunks * B

    mesh = plsc.VectorSubcoreMesh(core_axis_name="c", subcore_axis_name="s")

    @functools.partial(
        pl.kernel,
        mesh=mesh,
        out_type=jax.ShapeDtypeStruct((B * S * D,), jnp.float32),
        scratch_types=[
            pltpu.VMEM((2, CHW), jnp.float32),
            pltpu.VMEM((2, CHW), jnp.float32),
            pltpu.SemaphoreType.DMA,
            pltpu.SemaphoreType.DMA,
            pltpu.SemaphoreType.DMA,
        ],
    )
    def run(x_hbm, pos_hbm, out_hbm, p_v, x_v, sem_in, sem_pos, sem_out):
        wid = lax.axis_index("s") * NC + lax.axis_index("c")
        s_base = wid * SW

        def x_off(t):
            c, b = t // B, t % B
            return (b * S + s_base + c * C) * D

        def start_in(t):
            pltpu.async_copy(x_hbm.at[pl.ds(x_off(t), CHW)], x_v.at[t % 2], sem_in)

        def start_pos(c):
            pltpu.async_copy(
                pos_hbm.at[pl.ds((s_base + c * C) * D, CHW)], p_v.at[c % 2], sem_pos
            )

        def wait(src, dst, sem):
            pltpu.make_async_copy(src, dst, sem).wait()

        start_pos(0)
        start_in(0)
        for t in range(n_steps):
            c = t // B
            if t % B == 0 and c + 1 < n_chunks:
                start_pos(c + 1)
            if t % B == 0:
                wait(pos_hbm.at[pl.ds(0, CHW)], p_v.at[c % 2], sem_pos)
            wait(x_hbm.at[pl.ds(0, CHW)], x_v.at[t % 2], sem_in)
            if t + 1 < n_steps:
                if t >= 1:
                    wait(x_v.at[(t - 1) % 2], out_hbm.at[pl.ds(0, CHW)], sem_out)
                start_in(t + 1)

            xb = x_v.at[t % 2]
            pb = p_v.at[c % 2]

            def add_body(i, acc):
                base = i * 128
                for k in range(8):
                    off = base + k * 16
                    xb[pl.ds(off, 16)] = xb[pl.ds(off, 16)] + pb[pl.ds(off, 16)]
                return acc

            lax.fori_loop(0, CHW // 128, add_body, 0)

            pltpu.async_copy(xb, out_hbm.at[pl.ds(x_off(t), CHW)], sem_out)
        wait(x_v.at[0], out_hbm.at[pl.ds(0, CHW)], sem_out)
        wait(x_v.at[1], out_hbm.at[pl.ds(0, CHW)], sem_out)

    return run


def kernel(x, pos_weight):
    B, S, D = x.shape
    out = _sc_add(B, S, D)(x.reshape(-1), pos_weight[:S].reshape(-1))
    return out.reshape(B, S, D)
